# Initial kernel scaffold; baseline (speedup 1.0000x reference)
import functools

import jax
import jax.numpy as jnp
from jax import lax
from jax.experimental import pallas as pl
from jax.experimental.pallas import tpu as pltpu
from jax.experimental.pallas import tpu_sc as plsc

NC = 2
NS = 16
NW = NC * NS
CH = 128
DW = 16


def _matmul_kernel(x_ref, w_ref, b_ref, o_ref):
    o_ref[...] = (
        jnp.dot(x_ref[...], w_ref[...], preferred_element_type=jnp.float32)
        + b_ref[...]
    )


def _finalize_kernel(a_ref, d_ref, o_ref):
    ssum = a_ref[0] + a_ref[1]
    deg = d_ref[0, :, 0:1] + d_ref[1, :, 0:1]
    deg = jnp.maximum(deg, 1.0)
    o_ref[...] = jnp.maximum(ssum / deg, 0.0)


def _make_sc_agg(n_pad, d, nch):
    rpt = n_pad // (NS * CH)

    mesh = plsc.VectorSubcoreMesh(core_axis_name="c", subcore_axis_name="s")

    @functools.partial(
        pl.kernel,
        mesh=mesh,
        out_type=[
            jax.ShapeDtypeStruct((NC, n_pad, d), jnp.float32),
            jax.ShapeDtypeStruct((NC, n_pad, DW), jnp.float32),
        ],
        scratch_types=[
            pltpu.VMEM((nch, CH), jnp.int32),
            pltpu.VMEM((nch, CH), jnp.int32),
            pltpu.VMEM((CH, d), jnp.float32),
            pltpu.VMEM((CH, DW), jnp.float32),
            pltpu.VMEM_SHARED((n_pad, d), jnp.float32),
            pltpu.VMEM_SHARED((n_pad, DW), jnp.float32),
            pltpu.SemaphoreType.DMA,
        ],
    )
    def sc_agg(h_hbm, src_hbm, dst_hbm, zrow_hbm, zdeg_hbm, ones_hbm,
               agg_out, deg_out,
               src_v, dst_v, rows_v, ones_v, acc_sh, deg_sh, sem):
        c = lax.axis_index("c")
        s = lax.axis_index("s")

        def zbody(i, carry):
            r0 = (s * rpt + i) * CH
            pltpu.sync_copy(zrow_hbm, acc_sh.at[pl.ds(r0, CH)])
            pltpu.sync_copy(zdeg_hbm, deg_sh.at[pl.ds(r0, CH)])
            return carry

        lax.fori_loop(0, rpt, zbody, 0)

        pltpu.sync_copy(src_hbm.at[c, s], src_v)
        pltpu.sync_copy(dst_hbm.at[c, s], dst_v)
        pltpu.sync_copy(ones_hbm, ones_v)
        plsc.subcore_barrier()

        def ebody(j, carry):
            pltpu.async_copy(h_hbm.at[src_v.at[j]], rows_v, sem).wait()
            pltpu.sync_copy(rows_v, acc_sh.at[dst_v.at[j]], add=True)
            pltpu.sync_copy(ones_v, deg_sh.at[dst_v.at[j]], add=True)
            return carry

        lax.fori_loop(0, nch, ebody, 0)
        plsc.subcore_barrier()

        def wbody(i, carry):
            r0 = (s * rpt + i) * CH
            pltpu.sync_copy(acc_sh.at[pl.ds(r0, CH)],
                            agg_out.at[c, pl.ds(r0, CH)])
            pltpu.sync_copy(deg_sh.at[pl.ds(r0, CH)],
                            deg_out.at[c, pl.ds(r0, CH)])
            return carry

        lax.fori_loop(0, rpt, wbody, 0)

    return sc_agg


def kernel(x, edge_index, W, b):
    n, d_in = x.shape
    d = W.shape[1]
    e = edge_index.shape[1]

    bn = 1000
    h = pl.pallas_call(
        _matmul_kernel,
        grid=(n // bn,),
        in_specs=[
            pl.BlockSpec((bn, d_in), lambda i: (i, 0)),
            pl.BlockSpec((d_in, d), lambda i: (0, 0)),
            pl.BlockSpec((1, d), lambda i: (0, 0)),
        ],
        out_specs=pl.BlockSpec((bn, d), lambda i: (i, 0)),
        out_shape=jax.ShapeDtypeStruct((n, d), jnp.float32),
    )(x, W, b.reshape(1, d))

    nch = -(-e // (NW * CH))
    e_pad = NW * nch * CH
    n_pad = -(-(n + 1) // (NS * CH)) * (NS * CH)

    src = edge_index[0]
    dst = edge_index[1]
    pad = e_pad - e
    src3 = jnp.concatenate([src, jnp.zeros((pad,), jnp.int32)]).reshape(
        NC, NS, nch, CH)
    dst3 = jnp.concatenate([dst, jnp.full((pad,), n, jnp.int32)]).reshape(
        NC, NS, nch, CH)

    zrow = jnp.zeros((CH, d), jnp.float32)
    zdeg = jnp.zeros((CH, DW), jnp.float32)
    ones = jnp.ones((CH, DW), jnp.float32)

    agg_p, deg_p = _make_sc_agg(n_pad, d, nch)(h, src3, dst3, zrow, zdeg, ones)

    out = pl.pallas_call(
        _finalize_kernel,
        grid=(n // bn,),
        in_specs=[
            pl.BlockSpec((NC, bn, d), lambda i: (0, i, 0)),
            pl.BlockSpec((NC, bn, DW), lambda i: (0, i, 0)),
        ],
        out_specs=pl.BlockSpec((bn, d), lambda i: (i, 0)),
        out_shape=jax.ShapeDtypeStruct((n, d), jnp.float32),
    )(agg_p, deg_p)
    return out


# speedup vs baseline: 6.1575x; 6.1575x over previous
import functools

import jax
import jax.numpy as jnp
from jax import lax
from jax.experimental import pallas as pl
from jax.experimental.pallas import tpu as pltpu
from jax.experimental.pallas import tpu_sc as plsc

NC = 2
NS = 16
CH = 128
DW = 16


def _matmul_kernel(x_ref, w_ref, b_ref, o_ref):
    o_ref[0] = (
        jnp.dot(x_ref[...], w_ref[0], preferred_element_type=jnp.float32)
        + b_ref[0]
    )


def _finalize_kernel(a_ref, d_ref, o_ref):
    ssum = jnp.concatenate([a_ref[0], a_ref[1]], axis=-1)
    deg = jnp.maximum(d_ref[0, :, 0:1], 1.0)
    o_ref[...] = jnp.maximum(ssum / deg, 0.0)


def _make_sc_agg(n_pad, dh, nch):
    rpt = n_pad // (NS * CH)

    mesh = plsc.VectorSubcoreMesh(core_axis_name="c", subcore_axis_name="s")

    @functools.partial(
        pl.kernel,
        mesh=mesh,
        compiler_params=pltpu.CompilerParams(use_tc_tiling_on_sc=False),
        out_type=[
            jax.ShapeDtypeStruct((NC, n_pad, dh), jnp.float32),
            jax.ShapeDtypeStruct((NC, n_pad, DW), jnp.float32),
        ],
        scratch_types=[
            pltpu.VMEM((nch, CH), jnp.int32),
            pltpu.VMEM((nch, CH), jnp.int32),
            pltpu.VMEM((CH, dh), jnp.float32),
            pltpu.VMEM((CH, DW), jnp.float32),
            pltpu.VMEM_SHARED((n_pad, dh), jnp.float32),
            pltpu.VMEM_SHARED((n_pad, DW), jnp.float32),
            pltpu.SemaphoreType.DMA,
        ],
    )
    def sc_agg(h_hbm, src_hbm, dst_hbm, zrow_hbm, zdeg_hbm, ones_hbm,
               agg_out, deg_out,
               src_v, dst_v, rows_v, ones_v, acc_sh, deg_sh, sem):
        c = lax.axis_index("c")
        s = lax.axis_index("s")

        def zbody(i, carry):
            r0 = (s * rpt + i) * CH
            pltpu.sync_copy(zrow_hbm, acc_sh.at[pl.ds(r0, CH)])
            pltpu.sync_copy(zdeg_hbm, deg_sh.at[pl.ds(r0, CH)])
            return carry

        lax.fori_loop(0, rpt, zbody, 0)

        pltpu.sync_copy(src_hbm.at[s], src_v)
        pltpu.sync_copy(dst_hbm.at[s], dst_v)
        pltpu.sync_copy(ones_hbm, ones_v)
        plsc.subcore_barrier()

        def ebody(j, carry):
            pltpu.async_copy(h_hbm.at[c].at[src_v.at[j]], rows_v, sem).wait()
            pltpu.sync_copy(rows_v, acc_sh.at[dst_v.at[j]], add=True)
            pltpu.sync_copy(ones_v, deg_sh.at[dst_v.at[j]], add=True)
            return carry

        lax.fori_loop(0, nch, ebody, 0)
        plsc.subcore_barrier()

        def wbody(i, carry):
            r0 = (s * rpt + i) * CH
            pltpu.sync_copy(acc_sh.at[pl.ds(r0, CH)],
                            agg_out.at[c, pl.ds(r0, CH)])
            pltpu.sync_copy(deg_sh.at[pl.ds(r0, CH)],
                            deg_out.at[c, pl.ds(r0, CH)])
            return carry

        lax.fori_loop(0, rpt, wbody, 0)

    return sc_agg


def kernel(x, edge_index, W, b):
    n, d_in = x.shape
    d = W.shape[1]
    dh = d // NC
    e = edge_index.shape[1]

    bn = 1000
    h = pl.pallas_call(
        _matmul_kernel,
        grid=(NC, n // bn),
        in_specs=[
            pl.BlockSpec((bn, d_in), lambda c, i: (i, 0)),
            pl.BlockSpec((1, d_in, dh), lambda c, i: (c, 0, 0)),
            pl.BlockSpec((1, 1, dh), lambda c, i: (c, 0, 0)),
        ],
        out_specs=pl.BlockSpec((1, bn, dh), lambda c, i: (c, i, 0)),
        out_shape=jax.ShapeDtypeStruct((NC, n, dh), jnp.float32),
    )(x, W.reshape(d_in, NC, dh).swapaxes(0, 1), b.reshape(NC, 1, dh))

    nch = -(-e // (NS * CH))
    e_pad = NS * nch * CH
    n_pad = -(-(n + 1) // (NS * CH)) * (NS * CH)

    src = edge_index[0]
    dst = edge_index[1]
    pad = e_pad - e
    src3 = jnp.concatenate([src, jnp.zeros((pad,), jnp.int32)]).reshape(
        NS, nch, CH)
    dst3 = jnp.concatenate([dst, jnp.full((pad,), n, jnp.int32)]).reshape(
        NS, nch, CH)

    zrow = jnp.zeros((CH, dh), jnp.float32)
    zdeg = jnp.zeros((CH, DW), jnp.float32)
    ones = jnp.ones((CH, DW), jnp.float32)

    agg_p, deg_p = _make_sc_agg(n_pad, dh, nch)(h, src3, dst3, zrow, zdeg,
                                                ones)

    out = pl.pallas_call(
        _finalize_kernel,
        grid=(n // bn,),
        in_specs=[
            pl.BlockSpec((NC, bn, dh), lambda i: (0, i, 0)),
            pl.BlockSpec((1, bn, DW), lambda i: (0, i, 0)),
        ],
        out_specs=pl.BlockSpec((bn, d), lambda i: (i, 0)),
        out_shape=jax.ShapeDtypeStruct((n, d), jnp.float32),
    )(agg_p, deg_p)
    return out
